# baseline (device time: 31052 ns/iter reference)
import jax
import jax.numpy as jnp
from jax import lax
from jax.experimental import pallas as pl
from jax.experimental.pallas import tpu as pltpu

N_Z = 4
N_CHUNK = 4


def kernel(x, pi):
    m, rows, cols = x.shape
    q_rows = rows // 4
    c_rows = q_rows // N_CHUNK

    def body(pi_ref, x_ref, out_ref, send_buf,
             z_send, z_recv, x_send, x_recv,
             y_send, y_recv, d_send, d_recv, ack_sem):
        mx = lax.axis_index("x")
        my = lax.axis_index("y")
        mz = lax.axis_index("z")
        dst_z = pi_ref[mz]
        src_z = jnp.int32(0)
        for j in range(N_Z):
            src_z = lax.select(pi_ref[j] == mz, jnp.int32(j), src_z)

        q = 2 * mx + my

        send_buf[...] = x_ref[0, pl.ds(q * q_rows, q_rows), :].astype(jnp.bfloat16)

        barrier = pltpu.get_barrier_semaphore()
        for dev in ((mx, my, src_z), (mx, my, dst_z),
                    (1 - mx, my, mz), (mx, 1 - my, mz), (1 - mx, 1 - my, mz)):
            pl.semaphore_signal(barrier, inc=1, device_id=dev,
                                device_id_type=pl.DeviceIdType.MESH)
        pl.semaphore_wait(barrier, 5)

        def quarter_chunk(base_q, c):
            return out_ref.at[0, pl.ds(base_q * q_rows + c * c_rows, c_rows), :]

        z_rdmas = []
        for c in range(N_CHUNK):
            r = pltpu.make_async_remote_copy(
                src_ref=send_buf.at[pl.ds(c * c_rows, c_rows), :],
                dst_ref=quarter_chunk(q, c),
                send_sem=z_send.at[c],
                recv_sem=z_recv.at[c],
                device_id=(mx, my, dst_z),
                device_id_type=pl.DeviceIdType.MESH,
            )
            r.start()
            z_rdmas.append(r)

        plane_rdmas = []
        plane_targets = (
            ((1 - mx, my, mz), x_send, x_recv),
            ((mx, 1 - my, mz), y_send, y_recv),
            ((1 - mx, 1 - my, mz), d_send, d_recv),
        )
        for c in range(N_CHUNK):
            z_rdmas[c].wait_recv()
            for dev, s_sems, r_sems in plane_targets:
                r = pltpu.make_async_remote_copy(
                    src_ref=quarter_chunk(q, c),
                    dst_ref=quarter_chunk(q, c),
                    send_sem=s_sems.at[c],
                    recv_sem=r_sems.at[c],
                    device_id=dev,
                    device_id_type=pl.DeviceIdType.MESH,
                )
                r.start()
                plane_rdmas.append(r)
        pl.semaphore_signal(ack_sem, inc=1, device_id=(mx, my, src_z),
                            device_id_type=pl.DeviceIdType.MESH)

        for r in plane_rdmas:
            r.wait_recv()
        for r in z_rdmas:
            r.wait_send()
        for r in plane_rdmas:
            r.wait_send()
        pl.semaphore_wait(ack_sem, 1)

    return pl.pallas_call(
        body,
        out_shape=jax.ShapeDtypeStruct((m, rows, cols), jnp.bfloat16),
        in_specs=[
            pl.BlockSpec(memory_space=pltpu.SMEM),
            pl.BlockSpec(memory_space=pltpu.VMEM),
        ],
        out_specs=pl.BlockSpec(memory_space=pltpu.VMEM),
        scratch_shapes=[
            pltpu.VMEM((q_rows, cols), jnp.bfloat16),
            pltpu.SemaphoreType.DMA((N_CHUNK,)),
            pltpu.SemaphoreType.DMA((N_CHUNK,)),
            pltpu.SemaphoreType.DMA((N_CHUNK,)),
            pltpu.SemaphoreType.DMA((N_CHUNK,)),
            pltpu.SemaphoreType.DMA((N_CHUNK,)),
            pltpu.SemaphoreType.DMA((N_CHUNK,)),
            pltpu.SemaphoreType.DMA((N_CHUNK,)),
            pltpu.SemaphoreType.DMA((N_CHUNK,)),
            pltpu.SemaphoreType.REGULAR,
        ],
        compiler_params=pltpu.CompilerParams(collective_id=0),
    )(pi, x)


# device time: 30024 ns/iter; 1.0342x vs baseline; 1.0342x over previous
import jax
import jax.numpy as jnp
from jax import lax
from jax.experimental import pallas as pl
from jax.experimental.pallas import tpu as pltpu

N_Z = 4
N_CHUNK = 8


def kernel(x, pi):
    m, rows, cols = x.shape
    q_rows = rows // 4
    c_rows = q_rows // N_CHUNK

    def body(pi_ref, x_ref, out_ref, q_buf, send_buf, local_sem,
             z_send, z_recv, x_send, x_recv,
             y_send, y_recv, d_send, d_recv, ack_sem):
        mx = lax.axis_index("x")
        my = lax.axis_index("y")
        mz = lax.axis_index("z")
        dst_z = pi_ref[mz]
        src_z = jnp.int32(0)
        for j in range(N_Z):
            src_z = lax.select(pi_ref[j] == mz, jnp.int32(j), src_z)

        q = 2 * mx + my

        local_copy = pltpu.make_async_copy(
            x_ref.at[0, pl.ds(q * q_rows, q_rows), :], q_buf, local_sem)
        local_copy.start()

        barrier = pltpu.get_barrier_semaphore()
        for dev in ((mx, my, src_z), (mx, my, dst_z),
                    (1 - mx, my, mz), (mx, 1 - my, mz), (1 - mx, 1 - my, mz)):
            pl.semaphore_signal(barrier, inc=1, device_id=dev,
                                device_id_type=pl.DeviceIdType.MESH)

        local_copy.wait()
        send_buf[...] = q_buf[...].astype(jnp.bfloat16)

        pl.semaphore_wait(barrier, 5)

        def quarter_chunk(base_q, c):
            return out_ref.at[0, pl.ds(base_q * q_rows + c * c_rows, c_rows), :]

        z_rdmas = []
        for c in range(N_CHUNK):
            r = pltpu.make_async_remote_copy(
                src_ref=send_buf.at[pl.ds(c * c_rows, c_rows), :],
                dst_ref=quarter_chunk(q, c),
                send_sem=z_send.at[c],
                recv_sem=z_recv.at[c],
                device_id=(mx, my, dst_z),
                device_id_type=pl.DeviceIdType.MESH,
            )
            r.start()
            z_rdmas.append(r)

        plane_rdmas = []
        plane_targets = (
            ((1 - mx, my, mz), x_send, x_recv),
            ((mx, 1 - my, mz), y_send, y_recv),
            ((1 - mx, 1 - my, mz), d_send, d_recv),
        )
        for c in range(N_CHUNK):
            z_rdmas[c].wait_recv()
            for dev, s_sems, r_sems in plane_targets:
                r = pltpu.make_async_remote_copy(
                    src_ref=quarter_chunk(q, c),
                    dst_ref=quarter_chunk(q, c),
                    send_sem=s_sems.at[c],
                    recv_sem=r_sems.at[c],
                    device_id=dev,
                    device_id_type=pl.DeviceIdType.MESH,
                )
                r.start()
                plane_rdmas.append(r)
        pl.semaphore_signal(ack_sem, inc=1, device_id=(mx, my, src_z),
                            device_id_type=pl.DeviceIdType.MESH)

        for r in plane_rdmas:
            r.wait_recv()
        for r in z_rdmas:
            r.wait_send()
        for r in plane_rdmas:
            r.wait_send()
        pl.semaphore_wait(ack_sem, 1)

    return pl.pallas_call(
        body,
        out_shape=jax.ShapeDtypeStruct((m, rows, cols), jnp.bfloat16),
        in_specs=[
            pl.BlockSpec(memory_space=pltpu.SMEM),
            pl.BlockSpec(memory_space=pl.ANY),
        ],
        out_specs=pl.BlockSpec(memory_space=pltpu.VMEM),
        scratch_shapes=[
            pltpu.VMEM((q_rows, cols), x.dtype),
            pltpu.VMEM((q_rows, cols), jnp.bfloat16),
            pltpu.SemaphoreType.DMA,
            pltpu.SemaphoreType.DMA((N_CHUNK,)),
            pltpu.SemaphoreType.DMA((N_CHUNK,)),
            pltpu.SemaphoreType.DMA((N_CHUNK,)),
            pltpu.SemaphoreType.DMA((N_CHUNK,)),
            pltpu.SemaphoreType.DMA((N_CHUNK,)),
            pltpu.SemaphoreType.DMA((N_CHUNK,)),
            pltpu.SemaphoreType.DMA((N_CHUNK,)),
            pltpu.SemaphoreType.DMA((N_CHUNK,)),
            pltpu.SemaphoreType.REGULAR,
        ],
        compiler_params=pltpu.CompilerParams(collective_id=0),
    )(pi, x)


# device time: 29982 ns/iter; 1.0357x vs baseline; 1.0014x over previous
import jax
import jax.numpy as jnp
from jax import lax
from jax.experimental import pallas as pl
from jax.experimental.pallas import tpu as pltpu

N_Z = 4
N_CHUNK = 8


def kernel(x, pi):
    m, rows, cols = x.shape
    q_rows = rows // 4
    c_rows = q_rows // N_CHUNK

    def body(pi_ref, x_ref, out_ref, q_buf, send_buf, local_sem,
             z_send, z_recv, x_send, x_recv,
             y_send, y_recv, d_send, d_recv, ack_sem):
        mx = lax.axis_index("x")
        my = lax.axis_index("y")
        mz = lax.axis_index("z")
        dst_z = pi_ref[mz]
        src_z = jnp.int32(0)
        for j in range(N_Z):
            src_z = lax.select(pi_ref[j] == mz, jnp.int32(j), src_z)

        q = 2 * mx + my

        local_copy = pltpu.make_async_copy(
            x_ref.at[0, pl.ds(q * q_rows, q_rows), :], q_buf, local_sem)
        local_copy.start()

        barrier = pltpu.get_barrier_semaphore()
        for dev in ((mx, my, src_z), (mx, my, dst_z),
                    (1 - mx, my, mz), (mx, 1 - my, mz), (1 - mx, 1 - my, mz)):
            pl.semaphore_signal(barrier, inc=1, device_id=dev,
                                device_id_type=pl.DeviceIdType.MESH)

        local_copy.wait()
        send_buf[...] = q_buf[...].astype(jnp.bfloat16)

        pl.semaphore_wait(barrier, 5)

        def quarter_chunk(base_q, c):
            return out_ref.at[0, pl.ds(base_q * q_rows + c * c_rows, c_rows), :]

        z_rdmas = []
        for c in range(N_CHUNK):
            r = pltpu.make_async_remote_copy(
                src_ref=send_buf.at[pl.ds(c * c_rows, c_rows), :],
                dst_ref=quarter_chunk(q, c),
                send_sem=z_send.at[c],
                recv_sem=z_recv.at[c],
                device_id=(mx, my, dst_z),
                device_id_type=pl.DeviceIdType.MESH,
            )
            r.start()
            z_rdmas.append(r)

        plane_rdmas = []
        plane_targets = (
            ((1 - mx, my, mz), x_send, x_recv),
            ((mx, 1 - my, mz), y_send, y_recv),
            ((1 - mx, 1 - my, mz), d_send, d_recv),
        )
        for c in range(N_CHUNK):
            z_rdmas[c].wait_recv()
            for dev, s_sems, r_sems in plane_targets:
                r = pltpu.make_async_remote_copy(
                    src_ref=quarter_chunk(q, c),
                    dst_ref=quarter_chunk(q, c),
                    send_sem=s_sems.at[c],
                    recv_sem=r_sems.at[c],
                    device_id=dev,
                    device_id_type=pl.DeviceIdType.MESH,
                )
                r.start()
                plane_rdmas.append(r)
        pl.semaphore_signal(ack_sem, inc=1, device_id=(mx, my, src_z),
                            device_id_type=pl.DeviceIdType.MESH)

        for r in plane_rdmas:
            r.wait_recv()
        for r in z_rdmas:
            r.wait_send()
        for r in plane_rdmas:
            r.wait_send()
        pl.semaphore_wait(ack_sem, 1)

    return pl.pallas_call(
        body,
        out_shape=jax.ShapeDtypeStruct((m, rows, cols), jnp.bfloat16),
        in_specs=[
            pl.BlockSpec(memory_space=pltpu.SMEM),
            pl.BlockSpec(memory_space=pl.ANY),
        ],
        out_specs=pl.BlockSpec(memory_space=pl.ANY),
        scratch_shapes=[
            pltpu.VMEM((q_rows, cols), x.dtype),
            pltpu.VMEM((q_rows, cols), jnp.bfloat16),
            pltpu.SemaphoreType.DMA,
            pltpu.SemaphoreType.DMA((N_CHUNK,)),
            pltpu.SemaphoreType.DMA((N_CHUNK,)),
            pltpu.SemaphoreType.DMA((N_CHUNK,)),
            pltpu.SemaphoreType.DMA((N_CHUNK,)),
            pltpu.SemaphoreType.DMA((N_CHUNK,)),
            pltpu.SemaphoreType.DMA((N_CHUNK,)),
            pltpu.SemaphoreType.DMA((N_CHUNK,)),
            pltpu.SemaphoreType.DMA((N_CHUNK,)),
            pltpu.SemaphoreType.REGULAR,
        ],
        compiler_params=pltpu.CompilerParams(collective_id=0),
    )(pi, x)
